# baseline (speedup 1.0000x reference)
import functools

import jax
import jax.numpy as jnp
from jax import lax
from jax.experimental import pallas as pl
from jax.experimental.pallas import tpu as pltpu
from jax.experimental.pallas import tpu_sc as plsc

f32 = jnp.float32
i32 = jnp.int32

VOCAB = 65
T = 8
ROWB = 72
NKEY = ROWB * T
DPAD = 128
LSECOL = 65
LOSSC = 80
NTOK = 1024
NC = 2
NS = 16
NW = NC * NS
BPT = NTOK // NW


def _tc_body(tok_t_ref, pos_ref, w_ref, b_ref, ctab_ref):
    W = w_ref[...]
    Lt = lax.dot_general(tok_t_ref[...], W, (((0,), (0,)), ((), ())),
                         preferred_element_type=f32,
                         precision=lax.Precision.HIGHEST)
    Lt = Lt + b_ref[...]
    Lp = jnp.dot(pos_ref[...], W, preferred_element_type=f32,
                 precision=lax.Precision.HIGHEST)
    for t in range(T):
        blk = Lt + Lp[t:t + 1, :]
        m = jnp.max(blk, axis=1, keepdims=True)
        s = jnp.sum(jnp.exp(blk - m), axis=1, keepdims=True)
        ctab_ref[pl.ds(t * ROWB, VOCAB), :VOCAB] = blk
        ctab_ref[pl.ds(t * ROWB, VOCAB), LSECOL:LSECOL + 1] = m + jnp.log(s)


_tc_tables = pl.pallas_call(
    _tc_body,
    out_shape=jax.ShapeDtypeStruct((NKEY, DPAD), f32),
)


@functools.partial(
    pl.kernel,
    mesh=plsc.VectorSubcoreMesh(core_axis_name="c", subcore_axis_name="s"),
    out_type=jax.ShapeDtypeStruct((NTOK, DPAD), f32),
    scratch_types=[
        pltpu.VMEM((BPT,), i32),
        pltpu.VMEM((BPT,), i32),
        pltpu.VMEM((BPT,), i32),
        pltpu.VMEM((BPT,), i32),
        pltpu.VMEM((BPT, DPAD), f32),
        pltpu.SemaphoreType.DMA,
        pltpu.SemaphoreType.DMA,
        pltpu.SemaphoreType.DMA,
    ],
    compiler_params=pltpu.CompilerParams(
        needs_layout_passes=False, use_tc_tiling_on_sc=False),
)
def _sc_kernel(ctab, idxf, tgtf, out,
               idx_v, tgt_v, keys_v, orow_v, rows_v,
               sem, sem2, sem3):
    cid = lax.axis_index("c")
    sid = lax.axis_index("s")
    wid = sid * NC + cid
    tpos = lax.div(wid, 4)
    rbase = 32 * lax.rem(wid, 4)

    cp_idx = pltpu.async_copy(idxf.at[pl.ds(wid * BPT, BPT)], idx_v, sem)
    cp_tgt = pltpu.async_copy(tgtf.at[pl.ds(wid * BPT, BPT)], tgt_v, sem2)
    cp_idx.wait()

    lane = jnp.arange(16, dtype=i32)
    for c in range(BPT // 16):
        keys_v[pl.ds(c * 16, 16)] = tpos * ROWB + idx_v[pl.ds(c * 16, 16)]
        orow_v[pl.ds(c * 16, 16)] = (rbase + c * 16 + lane) * T + tpos

    pltpu.async_copy(ctab.at[keys_v], rows_v, sem3).wait()

    cp_tgt.wait()
    lsecol = jnp.full((16,), LSECOL, dtype=i32)
    acc = jnp.zeros((16,), f32)
    for c in range(BPT // 16):
        rowi = lane + c * 16
        lg = plsc.load_gather(rows_v, [rowi, lsecol])
        picked = plsc.load_gather(rows_v, [rowi, tgt_v[pl.ds(c * 16, 16)]])
        acc = acc + (lg - picked)
    rows_v[0, pl.ds(LOSSC, 16)] = acc * jnp.float32(1.0 / NTOK)
    pltpu.async_copy(rows_v, out.at[orow_v], sem3).wait()


def kernel(idx, targets, tok_table, pos_table, W, b):
    V = tok_table.shape[0]
    ctab = _tc_tables(tok_table.astype(f32).T, pos_table.astype(f32),
                      W.astype(f32), b.astype(f32))

    idxT = idx.astype(i32).T.reshape(-1)
    tgtT = targets.astype(i32).T.reshape(-1)
    out_pad = _sc_kernel(ctab, idxT, tgtT)

    logits = out_pad[:, :V]
    loss = jnp.sum(out_pad.reshape(4, 256, DPAD)[:, :T, LOSSC:LOSSC + 16])
    return (logits, loss)


# speedup vs baseline: 1.0415x; 1.0415x over previous
import functools

import jax
import jax.numpy as jnp
from jax import lax
from jax.experimental import pallas as pl
from jax.experimental.pallas import tpu as pltpu
from jax.experimental.pallas import tpu_sc as plsc

f32 = jnp.float32
i32 = jnp.int32

VOCAB = 65
T = 8
ROWB = 72
NKEY = ROWB * T
DPAD = 128
LSECOL = 65
LOSSC = 80
NTOK = 1024
NC = 1
NS = 16
NW = NC * NS
BPT = NTOK // NW


def _tc_body(tok_t_ref, pos_ref, w_ref, b_ref, ctab_ref):
    W = w_ref[...]
    Lt = lax.dot_general(tok_t_ref[...], W, (((0,), (0,)), ((), ())),
                         preferred_element_type=f32,
                         precision=lax.Precision.HIGHEST)
    Lt = Lt + b_ref[...]
    Lp = jnp.dot(pos_ref[...], W, preferred_element_type=f32,
                 precision=lax.Precision.HIGHEST)
    for t in range(T):
        blk = Lt + Lp[t:t + 1, :]
        m = jnp.max(blk, axis=1, keepdims=True)
        s = jnp.sum(jnp.exp(blk - m), axis=1, keepdims=True)
        ctab_ref[pl.ds(t * ROWB, VOCAB), :VOCAB] = blk
        ctab_ref[pl.ds(t * ROWB, VOCAB), LSECOL:LSECOL + 1] = m + jnp.log(s)


_tc_tables = pl.pallas_call(
    _tc_body,
    out_shape=jax.ShapeDtypeStruct((NKEY, DPAD), f32),
)


@functools.partial(
    pl.kernel,
    mesh=plsc.VectorSubcoreMesh(core_axis_name="c", subcore_axis_name="s",
                                num_cores=NC),
    out_type=jax.ShapeDtypeStruct((NTOK, DPAD), f32),
    scratch_types=[
        pltpu.VMEM((BPT,), i32),
        pltpu.VMEM((BPT,), i32),
        pltpu.VMEM((BPT,), i32),
        pltpu.VMEM((BPT,), i32),
        pltpu.VMEM((BPT, DPAD), f32),
        pltpu.SemaphoreType.DMA,
        pltpu.SemaphoreType.DMA,
        pltpu.SemaphoreType.DMA,
    ],
    compiler_params=pltpu.CompilerParams(
        needs_layout_passes=False, use_tc_tiling_on_sc=False),
)
def _sc_kernel(ctab, idxf, tgtf, out,
               idx_v, tgt_v, keys_v, orow_v, rows_v,
               sem, sem2, sem3):
    cid = lax.axis_index("c")
    sid = lax.axis_index("s")
    wid = sid * NC + cid
    tpos = lax.div(wid * BPT, 128)
    rbase = lax.rem(wid * BPT, 128)

    cp_idx = pltpu.async_copy(idxf.at[pl.ds(wid * BPT, BPT)], idx_v, sem)
    cp_tgt = pltpu.async_copy(tgtf.at[pl.ds(wid * BPT, BPT)], tgt_v, sem2)
    cp_idx.wait()

    lane = jnp.arange(16, dtype=i32)
    for c in range(BPT // 16):
        keys_v[pl.ds(c * 16, 16)] = tpos * ROWB + idx_v[pl.ds(c * 16, 16)]
        orow_v[pl.ds(c * 16, 16)] = (rbase + c * 16 + lane) * T + tpos

    pltpu.async_copy(ctab.at[keys_v], rows_v, sem3).wait()

    cp_tgt.wait()
    lsecol = jnp.full((16,), LSECOL, dtype=i32)
    acc = jnp.zeros((16,), f32)
    for c in range(BPT // 16):
        rowi = lane + c * 16
        lg = plsc.load_gather(rows_v, [rowi, lsecol])
        picked = plsc.load_gather(rows_v, [rowi, tgt_v[pl.ds(c * 16, 16)]])
        acc = acc + (lg - picked)
    rows_v[0, pl.ds(LOSSC, 16)] = acc * jnp.float32(1.0 / NTOK)
    pltpu.async_copy(rows_v, out.at[orow_v], sem3).wait()


def kernel(idx, targets, tok_table, pos_table, W, b):
    V = tok_table.shape[0]
    ctab = _tc_tables(tok_table.astype(f32).T, pos_table.astype(f32),
                      W.astype(f32), b.astype(f32))

    idxT = idx.astype(i32).T.reshape(-1)
    tgtT = targets.astype(i32).T.reshape(-1)
    out_pad = _sc_kernel(ctab, idxT, tgtT)

    logits = out_pad[:, :V]
    nb = NTOK // (BPT * T)
    loss = jnp.sum(out_pad.reshape(nb, NTOK // nb, DPAD)[:, :T,
                                   LOSSC:LOSSC + 16])
    return (logits, loss)
